# baseline (speedup 1.0000x reference)
import functools

import jax
import jax.numpy as jnp
from jax import lax
from jax.experimental import pallas as pl
from jax.experimental.pallas import tpu as pltpu
from jax.experimental.pallas import tpu_sc as plsc

NC = 2
NS = 16
NW = NC * NS
L = 16

NBUF = 4


def _gather_kernel(B0, S, d, b_per_w):
    assert b_per_w == 128 and d == 64
    KT = d // 8
    mesh = plsc.VectorSubcoreMesh(
        core_axis_name="c", subcore_axis_name="s", num_cores=NC, num_subcores=NS
    )

    @functools.partial(
        pl.kernel,
        mesh=mesh,
        compiler_params=pltpu.CompilerParams(
            use_tc_tiling_on_sc=False, needs_layout_passes=False
        ),
        out_type=jax.ShapeDtypeStruct((S, KT, NW, 8, 128), jnp.float32),
        scratch_types=[
            pltpu.VMEM((b_per_w, S), jnp.int32),
            pltpu.VMEM((S * b_per_w,), jnp.int32),
            pltpu.VMEM((NBUF, b_per_w, d), jnp.float32),
            pltpu.VMEM((NBUF, KT, 8, 128), jnp.float32),
            pltpu.SemaphoreType.DMA((NBUF,)),
            pltpu.SemaphoreType.DMA((NBUF,)),
        ],
    )
    def k(idx_hbm, table_hbm, out_hbm, idx_v, idx_t, rows_v, obuf, gsem, wsem):
        table_flat = table_hbm.at[0]
        wid = lax.axis_index("s") * NC + lax.axis_index("c")
        base_b = wid * b_per_w
        pltpu.sync_copy(idx_hbm.at[pl.ds(base_b, b_per_w)], idx_v)

        iota = lax.iota(jnp.int32, L)

        def tr_idx(s, carry):
            for g in range(b_per_w // L):
                bs = iota + (L * g)
                vals = plsc.load_gather(idx_v, [bs, jnp.full((L,), s, jnp.int32)])
                plsc.store_scatter(idx_t, [s * b_per_w + (L * g) + iota], vals)
            return carry

        lax.fori_loop(0, S, tr_idx, 0)

        def gather_desc(j, slot):
            return pltpu.make_async_copy(
                table_flat.at[idx_t.at[pl.ds(j * b_per_w, b_per_w)]],
                rows_v.at[slot],
                gsem.at[slot],
            )

        def write_desc(j, slot):
            return pltpu.make_async_copy(
                obuf.at[slot], out_hbm.at[j, :, wid], wsem.at[slot]
            )

        def transpose_chunk(u):
            def tr(kk, carry):
                for ks in range(8):
                    kf = jnp.full((L,), kk * 8 + ks, jnp.int32)
                    kkf = jnp.full((L,), kk, jnp.int32)
                    ksf = jnp.full((L,), ks, jnp.int32)
                    for g in range(b_per_w // L):
                        bs = iota + (L * g)
                        vals = plsc.load_gather(rows_v.at[u], [bs, kf])
                        plsc.store_scatter(obuf.at[u], [kkf, ksf, bs], vals)
                return carry

            lax.fori_loop(0, KT, tr, 0)

        for j0 in range(NBUF - 1):
            gather_desc(j0, j0).start()

        def body(g, carry):
            for u in range(NBUF):
                j = g * NBUF + u
                pu = (u + NBUF - 1) % NBUF
                gather_desc(j, u).wait()

                @pl.when(j > 1)
                def _retire_write():
                    write_desc(j - 2, (u + NBUF - 2) % NBUF).wait()

                @pl.when(j + NBUF - 1 < S)
                def _prefetch():
                    gather_desc(j + NBUF - 1, pu).start()

                transpose_chunk(u)
                write_desc(j, u).start()

            return carry

        lax.fori_loop(0, S // NBUF, body, 0)
        write_desc(S - 2, (S - 2) % NBUF).wait()
        write_desc(S - 1, (S - 1) % NBUF).wait()

    return k


def kernel(indices_, tables):
    num_pages, page_size, d = tables.shape
    B0, S = indices_.shape
    b_per_w = B0 // NW
    assert B0 % NW == 0 and S % NBUF == 0

    idx = indices_.astype(jnp.int32)
    r = _gather_kernel(B0, S, d, b_per_w)(idx, tables)
    return jnp.transpose(r, (2, 4, 0, 1, 3)).reshape(B0, S, d)


# speedup vs baseline: 1.2286x; 1.2286x over previous
import functools

import jax
import jax.numpy as jnp
from jax import lax
from jax.experimental import pallas as pl
from jax.experimental.pallas import tpu as pltpu
from jax.experimental.pallas import tpu_sc as plsc

NC = 2
NS = 16
NW = NC * NS
L = 16

NBUF = 4


def _gather_kernel(B0, S, d, b_per_w):
    assert b_per_w == 128 and d == 64
    KT = d // 8
    mesh = plsc.VectorSubcoreMesh(
        core_axis_name="c", subcore_axis_name="s", num_cores=NC, num_subcores=NS
    )

    @functools.partial(
        pl.kernel,
        mesh=mesh,
        compiler_params=pltpu.CompilerParams(
            use_tc_tiling_on_sc=False, needs_layout_passes=False
        ),
        out_type=jax.ShapeDtypeStruct((S, KT, NW, 8, 128), jnp.float32),
        scratch_types=[
            pltpu.VMEM((b_per_w, S), jnp.int32),
            pltpu.VMEM((S * b_per_w,), jnp.int32),
            pltpu.VMEM((NBUF, b_per_w, d), jnp.float32),
            pltpu.VMEM((NBUF, KT, 8, 128), jnp.float32),
            pltpu.SemaphoreType.DMA((NBUF,)),
            pltpu.SemaphoreType.DMA((NBUF,)),
        ],
    )
    def k(idx_hbm, table_hbm, out_hbm, idx_v, idx_t, rows_v, obuf, gsem, wsem):
        table_flat = table_hbm.at[0]
        wid = lax.axis_index("s") * NC + lax.axis_index("c")
        base_b = wid * b_per_w
        pltpu.sync_copy(idx_hbm.at[pl.ds(base_b, b_per_w)], idx_v)

        iota = lax.iota(jnp.int32, L)

        def tr_idx(s, carry):
            for g in range(b_per_w // L):
                bs = iota + (L * g)
                vals = plsc.load_gather(idx_v, [bs, jnp.full((L,), s, jnp.int32)])
                plsc.store_scatter(idx_t, [s * b_per_w + (L * g) + iota], vals)
            return carry

        lax.fori_loop(0, S, tr_idx, 0)

        def gather_desc(j, slot):
            return pltpu.make_async_copy(
                table_flat.at[idx_t.at[pl.ds(j * b_per_w, b_per_w)]],
                rows_v.at[slot],
                gsem.at[slot],
            )

        def write_desc(j, slot):
            return pltpu.make_async_copy(
                obuf.at[slot], out_hbm.at[j, :, wid], wsem.at[slot]
            )

        kg_vecs = [
            ((iota + L * kg) >> 3, (iota + L * kg) & 7) for kg in range(d // L)
        ]

        def transpose_chunk(u):
            def tr(bs, bsv):
                for kg, (kkv, ksv) in enumerate(kg_vecs):
                    vals = rows_v.at[u][bs, pl.ds(L * kg, L)]
                    plsc.store_scatter(obuf.at[u], [kkv, ksv, bsv], vals)
                return bsv + 1

            lax.fori_loop(0, b_per_w, tr, jnp.zeros((L,), jnp.int32))

        for j0 in range(NBUF - 1):
            gather_desc(j0, j0).start()

        def body(g, carry):
            for u in range(NBUF):
                j = g * NBUF + u
                pu = (u + NBUF - 1) % NBUF
                gather_desc(j, u).wait()

                @pl.when(j > 1)
                def _retire_write():
                    write_desc(j - 2, (u + NBUF - 2) % NBUF).wait()

                @pl.when(j + NBUF - 1 < S)
                def _prefetch():
                    gather_desc(j + NBUF - 1, pu).start()

                transpose_chunk(u)
                write_desc(j, u).start()

            return carry

        lax.fori_loop(0, S // NBUF, body, 0)
        write_desc(S - 2, (S - 2) % NBUF).wait()
        write_desc(S - 1, (S - 1) % NBUF).wait()

    return k


def kernel(indices_, tables):
    num_pages, page_size, d = tables.shape
    B0, S = indices_.shape
    b_per_w = B0 // NW
    assert B0 % NW == 0 and S % NBUF == 0

    idx = indices_.astype(jnp.int32)
    r = _gather_kernel(B0, S, d, b_per_w)(idx, tables)
    return jnp.transpose(r, (2, 4, 0, 1, 3)).reshape(B0, S, d)


# speedup vs baseline: 1.5059x; 1.2258x over previous
import functools

import jax
import jax.numpy as jnp
from jax import lax
from jax.experimental import pallas as pl
from jax.experimental.pallas import tpu as pltpu
from jax.experimental.pallas import tpu_sc as plsc

NC = 2
NS = 16
NW = NC * NS
L = 16

NBUF = 4


def _gather_kernel(B0, S, d, b_per_w):
    assert b_per_w == 128 and d == 64
    KT = d // 8
    mesh = plsc.VectorSubcoreMesh(
        core_axis_name="c", subcore_axis_name="s", num_cores=NC, num_subcores=NS
    )

    @functools.partial(
        pl.kernel,
        mesh=mesh,
        compiler_params=pltpu.CompilerParams(
            use_tc_tiling_on_sc=False, needs_layout_passes=False
        ),
        out_type=jax.ShapeDtypeStruct((S, KT, NW, 8, 128), jnp.float32),
        scratch_types=[
            pltpu.VMEM((b_per_w, S), jnp.int32),
            pltpu.VMEM((S * b_per_w,), jnp.int32),
            pltpu.VMEM((NBUF, b_per_w, d), jnp.float32),
            pltpu.VMEM((NBUF, KT, 8, 128), jnp.float32),
            pltpu.SemaphoreType.DMA((NBUF,)),
            pltpu.SemaphoreType.DMA((NBUF,)),
        ],
    )
    def k(idx_hbm, table_hbm, out_hbm, idx_v, idx_t, rows_v, obuf, gsem, wsem):
        table_flat = table_hbm.at[0]
        wid = lax.axis_index("s") * NC + lax.axis_index("c")
        base_b = wid * b_per_w
        pltpu.sync_copy(idx_hbm.at[pl.ds(base_b, b_per_w)], idx_v)

        iota = lax.iota(jnp.int32, L)

        @plsc.parallel_loop(0, S, unroll=8)
        def _tr_idx(s):
            sf = jnp.full((L,), s, jnp.int32)
            for g in range(b_per_w // L):
                bs = iota + (L * g)
                vals = plsc.load_gather(idx_v, [bs, sf])
                plsc.store_scatter(idx_t, [s * b_per_w + (L * g) + iota], vals)

        def gather_desc(j, slot):
            return pltpu.make_async_copy(
                table_flat.at[idx_t.at[pl.ds(j * b_per_w, b_per_w)]],
                rows_v.at[slot],
                gsem.at[slot],
            )

        def write_desc(j, slot):
            return pltpu.make_async_copy(
                obuf.at[slot], out_hbm.at[j, :, wid], wsem.at[slot]
            )

        kg_vecs = [
            ((iota + L * kg) >> 3, (iota + L * kg) & 7) for kg in range(d // L)
        ]

        def transpose_chunk(u):
            @plsc.parallel_loop(0, b_per_w, unroll=8)
            def _tr(bs):
                bsv = jnp.full((L,), bs, jnp.int32)
                for kg, (kkv, ksv) in enumerate(kg_vecs):
                    vals = rows_v.at[u][bs, pl.ds(L * kg, L)]
                    plsc.store_scatter(obuf.at[u], [kkv, ksv, bsv], vals)

        for j0 in range(NBUF - 1):
            gather_desc(j0, j0).start()

        def body(g, carry):
            for u in range(NBUF):
                j = g * NBUF + u
                pu = (u + NBUF - 1) % NBUF
                gather_desc(j, u).wait()

                @pl.when(j > 1)
                def _retire_write():
                    write_desc(j - 2, (u + NBUF - 2) % NBUF).wait()

                @pl.when(j + NBUF - 1 < S)
                def _prefetch():
                    gather_desc(j + NBUF - 1, pu).start()

                transpose_chunk(u)
                write_desc(j, u).start()

            return carry

        lax.fori_loop(0, S // NBUF, body, 0)
        write_desc(S - 2, (S - 2) % NBUF).wait()
        write_desc(S - 1, (S - 1) % NBUF).wait()

    return k


def kernel(indices_, tables):
    num_pages, page_size, d = tables.shape
    B0, S = indices_.shape
    b_per_w = B0 // NW
    assert B0 % NW == 0 and S % NBUF == 0

    idx = indices_.astype(jnp.int32)
    r = _gather_kernel(B0, S, d, b_per_w)(idx, tables)
    return jnp.transpose(r, (2, 4, 0, 1, 3)).reshape(B0, S, d)


# speedup vs baseline: 2.6574x; 1.7646x over previous
import functools

import jax
import jax.numpy as jnp
from jax import lax
from jax.experimental import pallas as pl
from jax.experimental.pallas import tpu as pltpu
from jax.experimental.pallas import tpu_sc as plsc

NC = 2
NS = 16
NW = NC * NS
L = 16

NBUF = 4


def _gather_kernel(B0, S, d, b_per_w):
    assert b_per_w == 128 and d == 64
    KT = d // 8
    mesh = plsc.VectorSubcoreMesh(
        core_axis_name="c", subcore_axis_name="s", num_cores=NC, num_subcores=NS
    )

    @functools.partial(
        pl.kernel,
        mesh=mesh,
        compiler_params=pltpu.CompilerParams(
            use_tc_tiling_on_sc=False, needs_layout_passes=False
        ),
        out_type=jax.ShapeDtypeStruct((S, KT, NW, 8, 128), jnp.float32),
        scratch_types=[
            pltpu.VMEM((b_per_w, S), jnp.int32),
            pltpu.VMEM((S * b_per_w,), jnp.int32),
            pltpu.VMEM((NBUF, b_per_w, d), jnp.float32),
            pltpu.VMEM((NBUF, KT, 8, 129), jnp.float32),
            pltpu.SemaphoreType.DMA((NBUF,)),
            pltpu.SemaphoreType.DMA((NBUF,)),
        ],
    )
    def k(idx_hbm, table_hbm, out_hbm, idx_v, idx_t, rows_v, obuf, gsem, wsem):
        table_flat = table_hbm.at[0]
        wid = lax.axis_index("s") * NC + lax.axis_index("c")
        base_b = wid * b_per_w
        pltpu.sync_copy(idx_hbm.at[pl.ds(base_b, b_per_w)], idx_v)

        iota = lax.iota(jnp.int32, L)

        @plsc.parallel_loop(0, S, unroll=8)
        def _tr_idx(s):
            sf = jnp.full((L,), s, jnp.int32)
            for g in range(b_per_w // L):
                bs = iota + (L * g)
                vals = plsc.load_gather(idx_v, [bs, sf])
                plsc.store_scatter(idx_t, [s * b_per_w + (L * g) + iota], vals)

        def gather_desc(j, slot):
            return pltpu.make_async_copy(
                table_flat.at[idx_t.at[pl.ds(j * b_per_w, b_per_w)]],
                rows_v.at[slot],
                gsem.at[slot],
            )

        def write_desc(j, slot):
            return pltpu.make_async_copy(
                obuf.at[slot].at[:, :, pl.ds(0, 128)],
                out_hbm.at[j, :, wid],
                wsem.at[slot],
            )

        kg_vecs = [
            ((iota + L * kg) >> 3, (iota + L * kg) & 7) for kg in range(d // L)
        ]

        def transpose_chunk(u):
            @plsc.parallel_loop(0, b_per_w, unroll=8)
            def _tr(bs):
                bsv = jnp.full((L,), bs, jnp.int32)
                for kg, (kkv, ksv) in enumerate(kg_vecs):
                    vals = rows_v.at[u][bs, pl.ds(L * kg, L)]
                    plsc.store_scatter(obuf.at[u], [kkv, ksv, bsv], vals)

        for j0 in range(NBUF - 1):
            gather_desc(j0, j0).start()

        def body(g, carry):
            for u in range(NBUF):
                j = g * NBUF + u
                pu = (u + NBUF - 1) % NBUF
                gather_desc(j, u).wait()

                @pl.when(j > 1)
                def _retire_write():
                    write_desc(j - 2, (u + NBUF - 2) % NBUF).wait()

                @pl.when(j + NBUF - 1 < S)
                def _prefetch():
                    gather_desc(j + NBUF - 1, pu).start()

                transpose_chunk(u)
                write_desc(j, u).start()

            return carry

        lax.fori_loop(0, S // NBUF, body, 0)
        write_desc(S - 2, (S - 2) % NBUF).wait()
        write_desc(S - 1, (S - 1) % NBUF).wait()

    return k


def kernel(indices_, tables):
    num_pages, page_size, d = tables.shape
    B0, S = indices_.shape
    b_per_w = B0 // NW
    assert B0 % NW == 0 and S % NBUF == 0

    idx = indices_.astype(jnp.int32)
    r = _gather_kernel(B0, S, d, b_per_w)(idx, tables)
    return jnp.transpose(r, (2, 4, 0, 1, 3)).reshape(B0, S, d)
